# Initial kernel scaffold; baseline (speedup 1.0000x reference)
import jax
import jax.numpy as jnp
from jax.experimental import pallas as pl


def kernel(x, W1, b1, Wg, bg, We, be, W2, b2):
    raise NotImplementedError("write your pallas kernel here")


# speedup vs baseline: 1.5612x; 1.5612x over previous
import jax
import jax.numpy as jnp
from jax.experimental import pallas as pl
from jax.experimental.pallas import tpu as pltpu

E = 8
TOPK = 2
D = 1024
N = 2048
BM = 256
NB = N // BM
NEG = -1e30


def _dense_body(x_ref, W1_ref, b1_ref, Wg_ref, bg_ref, We_ref, be_ref,
                W2_ref, b2_ref, out_ref, gstd_ref):
    i = pl.program_id(0)
    x = x_ref[...]
    h = jnp.maximum(
        jnp.dot(x, W1_ref[...], preferred_element_type=jnp.float32)
        + b1_ref[...], 0.0)
    logits = (jnp.dot(h, Wg_ref[...], preferred_element_type=jnp.float32)
              + bg_ref[...])
    ii = jax.lax.broadcasted_iota(jnp.int32, (BM, E), 1)
    m1 = jnp.max(logits, axis=1, keepdims=True)
    i1 = jnp.min(jnp.where(logits == m1, ii, E), axis=1, keepdims=True)
    lm = jnp.where(ii == i1, NEG, logits)
    m2 = jnp.max(lm, axis=1, keepdims=True)
    i2 = jnp.min(jnp.where(lm == m2, ii, E), axis=1, keepdims=True)
    w1 = 1.0 / (1.0 + jnp.exp(m2 - m1))
    w2 = 1.0 - w1

    g = jnp.exp(logits - m1)
    g = g / jnp.sum(g, axis=1, keepdims=True)
    mu = jnp.mean(g, axis=1, keepdims=True)
    var = jnp.sum((g - mu) ** 2, axis=1, keepdims=True) / (E - 1)
    part = jnp.sum(jnp.sqrt(var)) / N

    @pl.when(i == 0)
    def _():
        gstd_ref[...] = jnp.zeros_like(gstd_ref)

    gstd_ref[...] += jnp.reshape(part, (1, 1))

    be = be_ref[...]
    acc = jnp.zeros((BM, D), dtype=jnp.float32)
    for e in range(E):
        we = jnp.where(i1 == e, w1, 0.0) + jnp.where(i2 == e, w2, 0.0)
        acc = acc + jnp.dot(h * we, We_ref[e],
                            preferred_element_type=jnp.float32)
        acc = acc + we * be[e][None, :]
    moe = jnp.maximum(acc, 0.0)
    out = (jnp.dot(moe, W2_ref[...], preferred_element_type=jnp.float32)
           + b2_ref[...])
    out_ref[...] = out


def kernel(x, W1, b1, Wg, bg, We, be, W2, b2):
    out, gstd = pl.pallas_call(
        _dense_body,
        grid=(NB,),
        in_specs=[
            pl.BlockSpec((BM, D), lambda i: (i, 0)),
            pl.BlockSpec((D, D), lambda i: (0, 0)),
            pl.BlockSpec((1, D), lambda i: (0, 0)),
            pl.BlockSpec((D, E), lambda i: (0, 0)),
            pl.BlockSpec((1, E), lambda i: (0, 0)),
            pl.BlockSpec((E, D, D), lambda i: (0, 0, 0)),
            pl.BlockSpec((E, D), lambda i: (0, 0)),
            pl.BlockSpec((D, D), lambda i: (0, 0)),
            pl.BlockSpec((1, D), lambda i: (0, 0)),
        ],
        out_specs=[
            pl.BlockSpec((BM, D), lambda i: (i, 0)),
            pl.BlockSpec((1, 1), lambda i: (0, 0)),
        ],
        out_shape=[
            jax.ShapeDtypeStruct((N, D), jnp.float32),
            jax.ShapeDtypeStruct((1, 1), jnp.float32),
        ],
    )(x, W1, b1.reshape(1, D), Wg, bg.reshape(1, E), We, be, W2,
      b2.reshape(1, D))
    return out, gstd[0, 0]
